# Initial kernel scaffold; baseline (speedup 1.0000x reference)
import jax
import jax.numpy as jnp
from jax.experimental import pallas as pl


def kernel(x, l, center_img, center_skt):
    raise NotImplementedError("write your pallas kernel here")


# speedup vs baseline: 5.5413x; 5.5413x over previous
import jax
import jax.numpy as jnp
from jax import lax
from jax.experimental import pallas as pl
from jax.experimental.pallas import tpu as pltpu
from jax.experimental.pallas import tpu_sc as plsc

B = 4096
D = 384
C = 100000
MOM = 0.9

NTILES = 16
N_EL = B // NTILES
CH = 128
NSUB = N_EL // CH
LANES = 16


def _sc_phase(l_hbm, ci_hbm, cs_hbm,
              rep_o, r_o, cir_o, csr_o,
              l2d, vals2d, r2d, gbuf0, gbuf1, sem0, sem1):
    tid = lax.axis_index("s")
    base = tid * N_EL

    for c in range(NSUB):
        pltpu.sync_copy(l_hbm.at[pl.ds(base + c * CH, CH)], l2d.at[c])
    for c in range(NSUB):
        for k in range(CH // LANES):
            vals2d[c, pl.ds(k * LANES, LANES)] = (
                lax.iota(jnp.int32, LANES) + (base + c * CH + k * LANES))

    for c in range(NSUB):
        pltpu.sync_copy(vals2d.at[c], rep_o.at[l2d.at[c]])

    plsc.subcore_barrier()

    for c in range(NSUB):
        pltpu.sync_copy(rep_o.at[l2d.at[c]], r2d.at[c])
        pltpu.sync_copy(r2d.at[c], r_o.at[pl.ds(base + c * CH, CH)])

    cp0 = pltpu.async_copy(ci_hbm.at[l2d.at[0]], gbuf0, sem0)
    cp1 = pltpu.async_copy(cs_hbm.at[l2d.at[0]], gbuf1, sem1)
    cp0.wait()
    pltpu.sync_copy(gbuf0, cir_o.at[pl.ds(base, CH)])
    cp0 = pltpu.async_copy(ci_hbm.at[l2d.at[1]], gbuf0, sem0)
    cp1.wait()
    pltpu.sync_copy(gbuf1, csr_o.at[pl.ds(base, CH)])
    cp1 = pltpu.async_copy(cs_hbm.at[l2d.at[1]], gbuf1, sem1)
    cp0.wait()
    pltpu.sync_copy(gbuf0, cir_o.at[pl.ds(base + CH, CH)])
    cp1.wait()
    pltpu.sync_copy(gbuf1, csr_o.at[pl.ds(base + CH, CH)])


def _sc_call(l, ci, cs):
    mesh = plsc.VectorSubcoreMesh(core_axis_name="c", subcore_axis_name="s",
                                  num_cores=1)
    fn = pl.kernel(
        _sc_phase,
        out_type=(
            jax.ShapeDtypeStruct((C,), jnp.int32),
            jax.ShapeDtypeStruct((B,), jnp.int32),
            jax.ShapeDtypeStruct((B, D), jnp.float32),
            jax.ShapeDtypeStruct((B, D), jnp.float32),
        ),
        mesh=mesh,
        scratch_types=[
            pltpu.VMEM((NSUB, CH), jnp.int32),
            pltpu.VMEM((NSUB, CH), jnp.int32),
            pltpu.VMEM((NSUB, CH), jnp.int32),
            pltpu.VMEM((CH, D), jnp.float32),
            pltpu.VMEM((CH, D), jnp.float32),
            pltpu.SemaphoreType.DMA,
            pltpu.SemaphoreType.DMA,
        ],
    )
    _, r, cir, csr = fn(l, ci, cs)
    return r, cir, csr


RB = 512
NBLK = B // RB


def _tc_phase(rcol_ref, rrow_ref, x_ref, ci_ref, cs_ref, out_ref, sums):
    blk = pl.program_id(0)

    @pl.when(blk == 0)
    def _():
        sums[0] = 0.0
        sums[1] = 0.0

    rr = rcol_ref[...]
    mm = rr == rrow_ref[...]
    cnt = jnp.sum(mm.astype(jnp.float32), axis=1, keepdims=True)
    acc = jax.lax.dot_general(
        mm.astype(jnp.bfloat16), x_ref[...],
        (((1,), (0,)), ((), ())), preferred_element_type=jnp.float32)
    mean = acc * (1.0 / jnp.maximum(cnt, 1.0))
    upd = ci_ref[...] * MOM + mean * (1.0 - MOM)
    n2 = jnp.sum(upd * upd, axis=1, keepdims=True)
    uh = upd * jnp.where(n2 > 0, lax.rsqrt(n2), 1.0)
    diff = uh - cs_ref[...]
    d2 = jnp.sum(diff * diff, axis=1, keepdims=True)
    gidx = blk * RB + lax.broadcasted_iota(jnp.int32, (RB, 1), 0)
    m = rr == gidx
    sums[0] += jnp.sum(jnp.where(m, d2, 0.0))
    sums[1] += jnp.sum(jnp.where(m, 1.0, 0.0))

    @pl.when(blk == NBLK - 1)
    def _():
        out_ref[...] = (sums[0] / jnp.maximum(sums[1], 1.0)).reshape(1, 1)


def _tc_call(r, x_bf, cir, csr):
    return pl.pallas_call(
        _tc_phase,
        grid=(NBLK,),
        in_specs=[
            pl.BlockSpec((RB, 1), lambda i: (i, 0)),
            pl.BlockSpec((1, B), lambda i: (0, 0)),
            pl.BlockSpec((B, D), lambda i: (0, 0)),
            pl.BlockSpec((RB, D), lambda i: (i, 0)),
            pl.BlockSpec((RB, D), lambda i: (i, 0)),
        ],
        out_specs=pl.BlockSpec((1, 1), lambda i: (0, 0)),
        out_shape=jax.ShapeDtypeStruct((1, 1), jnp.float32),
        scratch_shapes=[pltpu.SMEM((2,), jnp.float32)],
    )(r.reshape(B, 1), r.reshape(1, B), x_bf, cir, csr)


def kernel(x, l, center_img, center_skt):
    r, cir, csr = _sc_call(l, center_img, center_skt)
    loss = _tc_call(r, x.astype(jnp.bfloat16), cir, csr)
    return loss.reshape(())
